# Initial kernel scaffold; baseline (speedup 1.0000x reference)
import jax
import jax.numpy as jnp
from jax.experimental import pallas as pl


def kernel(cls, regs, anchors):
    raise NotImplementedError("write your pallas kernel here")


# speedup vs baseline: 5.4145x; 5.4145x over previous
import functools
import math

import numpy as np
import jax
import jax.numpy as jnp
from jax import lax
from jax.experimental import pallas as pl

_NUM_PRE = 2000
_NUM_POST = 1000
_IOU_THR = 0.7
_MIN_SIZE = 16.0
_IMG_H = 800.0
_IMG_W = 800.0
_LOG_MAX = math.log(1000.0 / 16.0)

_N = 7500
_NPAD = 7680
_ROWS = _NPAD // 128
_OROWS = 1024 // 128

_INT_MIN = np.int32(-2147483648)


def _sortable_key(score):
    b = lax.bitcast_convert_type(score, jnp.int32)
    return jnp.where(b >= 0, b, _INT_MIN - b)


def _rpn_kernel(score_ref, regs_ref, anch_ref, out_ref):
    score = score_ref[0]
    dx = regs_ref[0, 0]
    dy = regs_ref[0, 1]
    dh = jnp.minimum(regs_ref[0, 2], _LOG_MAX)
    dw = jnp.minimum(regs_ref[0, 3], _LOG_MAX)
    ax1 = anch_ref[0, 0]
    ay1 = anch_ref[0, 1]
    ax2 = anch_ref[0, 2]
    ay2 = anch_ref[0, 3]

    rows = lax.broadcasted_iota(jnp.int32, (_ROWS, 128), 0)
    lanes = lax.broadcasted_iota(jnp.int32, (_ROWS, 128), 1)
    idx = rows * 128 + lanes

    key = _sortable_key(score)
    hi16 = jnp.right_shift(key, 16)
    lo16 = jnp.bitwise_and(key, jnp.int32(0xFFFF))

    h = jnp.int32(-32768)
    for bit in range(15, -1, -1):
        trial = h + jnp.int32(1 << bit)
        c = jnp.sum((hi16 >= trial).astype(jnp.int32))
        h = jnp.where(c >= _NUM_PRE, trial, h)

    g = jnp.sum((hi16 > h).astype(jnp.int32))
    m1 = jnp.int32(_NUM_PRE) - g
    eq_hi = hi16 == h
    l = jnp.int32(0)
    for bit in range(15, -1, -1):
        trial = l + jnp.int32(1 << bit)
        c = jnp.sum((eq_hi & (lo16 >= trial)).astype(jnp.int32))
        l = jnp.where(c >= m1, trial, l)

    kstar = jnp.bitwise_or(lax.shift_left(h, 16), l)
    gt = jnp.sum((key > kstar).astype(jnp.int32))
    m2 = jnp.int32(_NUM_PRE) - gt
    eq = key == kstar
    t = jnp.int32(0)
    for bit in range(12, -1, -1):
        trial = t + jnp.int32(1 << bit)
        c = jnp.sum((eq & (idx < trial)).astype(jnp.int32))
        t = jnp.where(c < m2, trial, t)

    sel = (key > kstar) | (eq & (idx <= t))

    ah = ay2 - ay1
    aw = ax2 - ax1
    cx = aw * 0.5
    cy = ah * 0.5
    px = cx + dx * aw
    py = cy + dy * ah
    ph = jnp.exp(dh) * ah
    pw = jnp.exp(dw) * aw
    bx1 = px - pw * 0.5
    by1 = py - ph * 0.5
    bx2 = px + pw * 0.5
    by2 = py + ph * 0.5

    cminx = jnp.clip(bx1, 0.0, _IMG_W)
    cminy = jnp.clip(by1, 0.0, _IMG_H)
    cmaxx = jnp.clip(bx2, 0.0, _IMG_W)
    cmaxy = jnp.clip(by2, 0.0, _IMG_H)
    size_ok = ((cmaxx - cminx) >= _MIN_SIZE) & ((cmaxy - cminy) >= _MIN_SIZE)

    s0 = jnp.where(sel & size_ok, score, -1e9)
    areas = (bx2 - bx1) * (by2 - by1)

    oidx = lax.broadcasted_iota(jnp.int32, (_OROWS, 128), 0) * 128 + \
        lax.broadcasted_iota(jnp.int32, (_OROWS, 128), 1)
    zero_plane = jnp.zeros((_OROWS, 128), jnp.float32)

    def body(i, state):
        s, o1, o2, o3, o4 = state
        m = jnp.max(s)
        j = jnp.min(jnp.where(s == m, idx, jnp.int32(1 << 30)))
        valid = (m > -1e8).astype(jnp.float32)
        onehot = idx == j
        jx1 = jnp.sum(jnp.where(onehot, bx1, 0.0))
        jy1 = jnp.sum(jnp.where(onehot, by1, 0.0))
        jx2 = jnp.sum(jnp.where(onehot, bx2, 0.0))
        jy2 = jnp.sum(jnp.where(onehot, by2, 0.0))
        jarea = (jx2 - jx1) * (jy2 - jy1)
        iw = jnp.maximum(jnp.minimum(jx2, bx2) - jnp.maximum(jx1, bx1), 0.0)
        ih = jnp.maximum(jnp.minimum(jy2, by2) - jnp.maximum(jy1, by1), 0.0)
        inter = iw * ih
        iou = inter / (jarea + areas - inter + 1e-9)
        s = jnp.where((iou > _IOU_THR) | onehot, -1e9, s)
        osel = oidx == i
        o1 = jnp.where(osel, jx1 * valid, o1)
        o2 = jnp.where(osel, jy1 * valid, o2)
        o3 = jnp.where(osel, jx2 * valid, o3)
        o4 = jnp.where(osel, jy2 * valid, o4)
        return (s, o1, o2, o3, o4)

    init = (s0, zero_plane, zero_plane, zero_plane, zero_plane)
    _, o1, o2, o3, o4 = lax.fori_loop(0, _NUM_POST, body, init)
    out_ref[0, 0] = o1
    out_ref[0, 1] = o2
    out_ref[0, 2] = o3
    out_ref[0, 3] = o4


@jax.jit
def kernel(cls, regs, anchors):
    B = cls.shape[0]
    score = jnp.transpose(cls, (0, 2, 3, 1)).reshape(B, -1)
    regs_f = jnp.transpose(regs, (0, 2, 3, 1)).reshape(B, -1, 4)

    pad = _NPAD - score.shape[1]
    score_p = jnp.pad(score, ((0, 0), (0, pad)),
                      constant_values=-jnp.inf).reshape(B, _ROWS, 128)
    regs_p = jnp.moveaxis(jnp.pad(regs_f, ((0, 0), (0, pad), (0, 0))), 2, 1)
    regs_p = regs_p.reshape(B, 4, _ROWS, 128)
    anch_p = jnp.moveaxis(jnp.pad(anchors, ((0, 0), (0, pad), (0, 0))), 2, 1)
    anch_p = anch_p.reshape(B, 4, _ROWS, 128)

    out = pl.pallas_call(
        _rpn_kernel,
        grid=(B,),
        in_specs=[
            pl.BlockSpec((1, _ROWS, 128), lambda b: (b, 0, 0)),
            pl.BlockSpec((1, 4, _ROWS, 128), lambda b: (b, 0, 0, 0)),
            pl.BlockSpec((1, 4, _ROWS, 128), lambda b: (b, 0, 0, 0)),
        ],
        out_specs=pl.BlockSpec((1, 4, _OROWS, 128), lambda b: (b, 0, 0, 0)),
        out_shape=jax.ShapeDtypeStruct((B, 4, _OROWS, 128), jnp.float32),
    )(score_p, regs_p, anch_p)

    boxes = jnp.moveaxis(out.reshape(B, 4, _OROWS * 128), 1, 2)
    return boxes[:, :_NUM_POST, :].reshape(B * _NUM_POST, 4)


# speedup vs baseline: 14.6638x; 2.7082x over previous
import functools
import math

import numpy as np
import jax
import jax.numpy as jnp
from jax import lax
from jax.experimental import pallas as pl

_NUM_PRE = 2000
_NUM_POST = 1000
_IOU_THR = 0.7
_MIN_SIZE = 16.0
_IMG_H = 800.0
_IMG_W = 800.0
_LOG_MAX = math.log(1000.0 / 16.0)

_N = 7500
_NPAD = 7680
_ROWS = _NPAD // 128
_OROWS = 1024 // 128

_INT_MIN = np.int32(-2147483648)


def _sortable_key(score):
    b = lax.bitcast_convert_type(score, jnp.int32)
    return jnp.where(b >= 0, b, _INT_MIN - b)


def _rpn_kernel(score_ref, regs_ref, anch_ref, out_ref):
    score = score_ref[0]
    dx = regs_ref[0, 0]
    dy = regs_ref[0, 1]
    dh = jnp.minimum(regs_ref[0, 2], _LOG_MAX)
    dw = jnp.minimum(regs_ref[0, 3], _LOG_MAX)
    ax1 = anch_ref[0, 0]
    ay1 = anch_ref[0, 1]
    ax2 = anch_ref[0, 2]
    ay2 = anch_ref[0, 3]

    rows = lax.broadcasted_iota(jnp.int32, (_ROWS, 128), 0)
    lanes = lax.broadcasted_iota(jnp.int32, (_ROWS, 128), 1)
    idx = rows * 128 + lanes

    key = _sortable_key(score)
    hi16 = jnp.right_shift(key, 16)
    lo16 = jnp.bitwise_and(key, jnp.int32(0xFFFF))

    h = jnp.int32(-32768)
    for bit in range(15, -1, -1):
        trial = h + jnp.int32(1 << bit)
        c = jnp.sum((hi16 >= trial).astype(jnp.int32))
        h = jnp.where(c >= _NUM_PRE, trial, h)

    g = jnp.sum((hi16 > h).astype(jnp.int32))
    m1 = jnp.int32(_NUM_PRE) - g
    eq_hi = hi16 == h
    l = jnp.int32(0)
    for bit in range(15, -1, -1):
        trial = l + jnp.int32(1 << bit)
        c = jnp.sum((eq_hi & (lo16 >= trial)).astype(jnp.int32))
        l = jnp.where(c >= m1, trial, l)

    kstar = jnp.bitwise_or(lax.shift_left(h, 16), l)
    gt = jnp.sum((key > kstar).astype(jnp.int32))
    m2 = jnp.int32(_NUM_PRE) - gt
    eq = key == kstar
    t = jnp.int32(0)
    for bit in range(12, -1, -1):
        trial = t + jnp.int32(1 << bit)
        c = jnp.sum((eq & (idx < trial)).astype(jnp.int32))
        t = jnp.where(c < m2, trial, t)

    sel = (key > kstar) | (eq & (idx <= t))

    ah = ay2 - ay1
    aw = ax2 - ax1
    cx = aw * 0.5
    cy = ah * 0.5
    px = cx + dx * aw
    py = cy + dy * ah
    ph = jnp.exp(dh) * ah
    pw = jnp.exp(dw) * aw
    bx1 = px - pw * 0.5
    by1 = py - ph * 0.5
    bx2 = px + pw * 0.5
    by2 = py + ph * 0.5

    cminx = jnp.clip(bx1, 0.0, _IMG_W)
    cminy = jnp.clip(by1, 0.0, _IMG_H)
    cmaxx = jnp.clip(bx2, 0.0, _IMG_W)
    cmaxy = jnp.clip(by2, 0.0, _IMG_H)
    size_ok = ((cmaxx - cminx) >= _MIN_SIZE) & ((cmaxy - cminy) >= _MIN_SIZE)

    s0 = jnp.where(sel & size_ok, score, -1e9)
    areas = (bx2 - bx1) * (by2 - by1)

    oidx = lax.broadcasted_iota(jnp.int32, (_OROWS, 128), 0) * 128 + \
        lax.broadcasted_iota(jnp.int32, (_OROWS, 128), 1)
    zero_plane = jnp.zeros((_OROWS, 128), jnp.float32)

    def cond(state):
        i, s, *_ = state
        return (i < _NUM_POST) & (jnp.max(s) > -1e8)

    def body(state):
        i, s, o1, o2, o3, o4 = state
        m = jnp.max(s)
        j = jnp.min(jnp.where(s == m, idx, jnp.int32(1 << 30)))
        onehot = idx == j
        jx1 = jnp.sum(jnp.where(onehot, bx1, 0.0))
        jy1 = jnp.sum(jnp.where(onehot, by1, 0.0))
        jx2 = jnp.sum(jnp.where(onehot, bx2, 0.0))
        jy2 = jnp.sum(jnp.where(onehot, by2, 0.0))
        jarea = (jx2 - jx1) * (jy2 - jy1)
        iw = jnp.maximum(jnp.minimum(jx2, bx2) - jnp.maximum(jx1, bx1), 0.0)
        ih = jnp.maximum(jnp.minimum(jy2, by2) - jnp.maximum(jy1, by1), 0.0)
        inter = iw * ih
        iou = inter / (jarea + areas - inter + 1e-9)
        s = jnp.where((iou > _IOU_THR) | onehot, -1e9, s)
        osel = oidx == i
        o1 = jnp.where(osel, jx1, o1)
        o2 = jnp.where(osel, jy1, o2)
        o3 = jnp.where(osel, jx2, o3)
        o4 = jnp.where(osel, jy2, o4)
        return (i + 1, s, o1, o2, o3, o4)

    init = (jnp.int32(0), s0, zero_plane, zero_plane, zero_plane, zero_plane)
    _, _, o1, o2, o3, o4 = lax.while_loop(cond, body, init)
    out_ref[0, 0] = o1
    out_ref[0, 1] = o2
    out_ref[0, 2] = o3
    out_ref[0, 3] = o4


@jax.jit
def kernel(cls, regs, anchors):
    B = cls.shape[0]
    score = jnp.transpose(cls, (0, 2, 3, 1)).reshape(B, -1)
    regs_f = jnp.transpose(regs, (0, 2, 3, 1)).reshape(B, -1, 4)

    pad = _NPAD - score.shape[1]
    score_p = jnp.pad(score, ((0, 0), (0, pad)),
                      constant_values=-jnp.inf).reshape(B, _ROWS, 128)
    regs_p = jnp.moveaxis(jnp.pad(regs_f, ((0, 0), (0, pad), (0, 0))), 2, 1)
    regs_p = regs_p.reshape(B, 4, _ROWS, 128)
    anch_p = jnp.moveaxis(jnp.pad(anchors, ((0, 0), (0, pad), (0, 0))), 2, 1)
    anch_p = anch_p.reshape(B, 4, _ROWS, 128)

    out = pl.pallas_call(
        _rpn_kernel,
        grid=(B,),
        in_specs=[
            pl.BlockSpec((1, _ROWS, 128), lambda b: (b, 0, 0)),
            pl.BlockSpec((1, 4, _ROWS, 128), lambda b: (b, 0, 0, 0)),
            pl.BlockSpec((1, 4, _ROWS, 128), lambda b: (b, 0, 0, 0)),
        ],
        out_specs=pl.BlockSpec((1, 4, _OROWS, 128), lambda b: (b, 0, 0, 0)),
        out_shape=jax.ShapeDtypeStruct((B, 4, _OROWS, 128), jnp.float32),
    )(score_p, regs_p, anch_p)

    boxes = jnp.moveaxis(out.reshape(B, 4, _OROWS * 128), 1, 2)
    return boxes[:, :_NUM_POST, :].reshape(B * _NUM_POST, 4)


# speedup vs baseline: 19.8074x; 1.3508x over previous
import math

import numpy as np
import jax
import jax.numpy as jnp
from jax import lax
from jax.experimental import pallas as pl

_NUM_PRE = 2000
_NUM_POST = 1000
_IOU_THR = 0.7
_MIN_SIZE = 16.0
_IMG_H = 800.0
_IMG_W = 800.0
_LOG_MAX = math.log(1000.0 / 16.0)

_NPAD = 7680
_ROWS = _NPAD // 128
_OROWS = 1024 // 128

_INT_MIN = np.int32(-2147483648)


def _sortable_key(score):
    b = lax.bitcast_convert_type(score, jnp.int32)
    return jnp.where(b >= 0, b, _INT_MIN - b)


def _top2000_mask(score, idx):
    key = _sortable_key(score)
    hi16 = jnp.right_shift(key, 16)
    lo16 = jnp.bitwise_and(key, jnp.int32(0xFFFF))

    h = jnp.int32(-32768)
    for bit in range(15, -1, -1):
        trial = h + jnp.int32(1 << bit)
        c = jnp.sum((hi16 >= trial).astype(jnp.int32))
        h = jnp.where(c >= _NUM_PRE, trial, h)

    g = jnp.sum((hi16 > h).astype(jnp.int32))
    m1 = jnp.int32(_NUM_PRE) - g
    eq_hi = hi16 == h
    l = jnp.int32(0)
    for bit in range(15, -1, -1):
        trial = l + jnp.int32(1 << bit)
        c = jnp.sum((eq_hi & (lo16 >= trial)).astype(jnp.int32))
        l = jnp.where(c >= m1, trial, l)

    kstar = jnp.bitwise_or(lax.shift_left(h, 16), l)
    gt = jnp.sum((key > kstar).astype(jnp.int32))
    m2 = jnp.int32(_NUM_PRE) - gt
    eq = key == kstar
    t = jnp.int32(0)
    for bit in range(12, -1, -1):
        trial = t + jnp.int32(1 << bit)
        c = jnp.sum((eq & (idx < trial)).astype(jnp.int32))
        t = jnp.where(c < m2, trial, t)

    return (key > kstar) | (eq & (idx <= t))


def _rpn_kernel(score_ref, regs_ref, anch_ref, out_ref):
    idx = lax.broadcasted_iota(jnp.int32, (_ROWS, 128), 0) * 128 + \
        lax.broadcasted_iota(jnp.int32, (_ROWS, 128), 1)
    oidx = lax.broadcasted_iota(jnp.int32, (_OROWS, 128), 0) * 128 + \
        lax.broadcasted_iota(jnp.int32, (_OROWS, 128), 1)
    zero_plane = jnp.zeros((_OROWS, 128), jnp.float32)

    s0 = []
    boxes = []
    for b in range(2):
        score = score_ref[b]
        dx = regs_ref[b, 0]
        dy = regs_ref[b, 1]
        dh = jnp.minimum(regs_ref[b, 2], _LOG_MAX)
        dw = jnp.minimum(regs_ref[b, 3], _LOG_MAX)
        ax1 = anch_ref[b, 0]
        ay1 = anch_ref[b, 1]
        ax2 = anch_ref[b, 2]
        ay2 = anch_ref[b, 3]

        sel = _top2000_mask(score, idx)

        ah = ay2 - ay1
        aw = ax2 - ax1
        px = aw * 0.5 + dx * aw
        py = ah * 0.5 + dy * ah
        ph = jnp.exp(dh) * ah
        pw = jnp.exp(dw) * aw
        bx1 = px - pw * 0.5
        by1 = py - ph * 0.5
        bx2 = px + pw * 0.5
        by2 = py + ph * 0.5

        cminx = jnp.clip(bx1, 0.0, _IMG_W)
        cminy = jnp.clip(by1, 0.0, _IMG_H)
        cmaxx = jnp.clip(bx2, 0.0, _IMG_W)
        cmaxy = jnp.clip(by2, 0.0, _IMG_H)
        size_ok = ((cmaxx - cminx) >= _MIN_SIZE) & ((cmaxy - cminy) >= _MIN_SIZE)

        s0.append(jnp.where(sel & size_ok, score, -1e9))
        area = (bx2 - bx1) * (by2 - by1)
        boxes.append((bx1, by1, bx2, by2, area))

    def peak(s):
        m = jnp.max(s)
        j = jnp.min(jnp.where(s == m, idx, jnp.int32(1 << 30)))
        return m, j

    def cond(state):
        i, ma, _, mb, _, _, _, _ = state
        return (i < _NUM_POST) & ((ma > -1e8) | (mb > -1e8))

    def step(i, m, j, s, outs, bx):
        bx1, by1, bx2, by2, area = bx
        valid = m > -1e8
        onehot = idx == j
        jx1 = jnp.sum(jnp.where(onehot, bx1, 0.0))
        jy1 = jnp.sum(jnp.where(onehot, by1, 0.0))
        jx2 = jnp.sum(jnp.where(onehot, bx2, 0.0))
        jy2 = jnp.sum(jnp.where(onehot, by2, 0.0))
        jarea = (jx2 - jx1) * (jy2 - jy1)
        iw = jnp.maximum(jnp.minimum(jx2, bx2) - jnp.maximum(jx1, bx1), 0.0)
        ih = jnp.maximum(jnp.minimum(jy2, by2) - jnp.maximum(jy1, by1), 0.0)
        inter = iw * ih
        iou = inter / (jarea + area - inter + 1e-9)
        s = jnp.where(((iou > _IOU_THR) | onehot) & valid, -1e9, s)
        osel = (oidx == i) & valid
        o1, o2, o3, o4 = outs
        o1 = jnp.where(osel, jx1, o1)
        o2 = jnp.where(osel, jy1, o2)
        o3 = jnp.where(osel, jx2, o3)
        o4 = jnp.where(osel, jy2, o4)
        m2, j2 = peak(s)
        return m2, j2, s, (o1, o2, o3, o4)

    def body(state):
        i, ma, ja, mb, jb, sa, sb, (oa, ob) = state
        ma, ja, sa, oa = step(i, ma, ja, sa, oa, boxes[0])
        mb, jb, sb, ob = step(i, mb, jb, sb, ob, boxes[1])
        return (i + 1, ma, ja, mb, jb, sa, sb, (oa, ob))

    ma0, ja0 = peak(s0[0])
    mb0, jb0 = peak(s0[1])
    zouts = (zero_plane, zero_plane, zero_plane, zero_plane)
    init = (jnp.int32(0), ma0, ja0, mb0, jb0, s0[0], s0[1], (zouts, zouts))
    final = lax.while_loop(cond, body, init)
    (oa1, oa2, oa3, oa4), (ob1, ob2, ob3, ob4) = final[7]
    out_ref[0, 0] = oa1
    out_ref[0, 1] = oa2
    out_ref[0, 2] = oa3
    out_ref[0, 3] = oa4
    out_ref[1, 0] = ob1
    out_ref[1, 1] = ob2
    out_ref[1, 2] = ob3
    out_ref[1, 3] = ob4


@jax.jit
def kernel(cls, regs, anchors):
    B = cls.shape[0]
    score = jnp.transpose(cls, (0, 2, 3, 1)).reshape(B, -1)
    regs_f = jnp.transpose(regs, (0, 2, 3, 1)).reshape(B, -1, 4)

    pad = _NPAD - score.shape[1]
    score_p = jnp.pad(score, ((0, 0), (0, pad)),
                      constant_values=-jnp.inf).reshape(B, _ROWS, 128)
    regs_p = jnp.moveaxis(jnp.pad(regs_f, ((0, 0), (0, pad), (0, 0))), 2, 1)
    regs_p = regs_p.reshape(B, 4, _ROWS, 128)
    anch_p = jnp.moveaxis(jnp.pad(anchors, ((0, 0), (0, pad), (0, 0))), 2, 1)
    anch_p = anch_p.reshape(B, 4, _ROWS, 128)

    out = pl.pallas_call(
        _rpn_kernel,
        out_shape=jax.ShapeDtypeStruct((B, 4, _OROWS, 128), jnp.float32),
    )(score_p, regs_p, anch_p)

    boxes = jnp.moveaxis(out.reshape(B, 4, _OROWS * 128), 1, 2)
    return boxes[:, :_NUM_POST, :].reshape(B * _NUM_POST, 4)


# speedup vs baseline: 20.1825x; 1.0189x over previous
import math

import numpy as np
import jax
import jax.numpy as jnp
from jax import lax
from jax.experimental import pallas as pl
from jax.experimental.pallas import tpu as pltpu

_NUM_PRE = 2000
_NUM_POST = 1000
_IOU_THR = 0.7
_MIN_SIZE = 16.0
_IMG_H = 800.0
_IMG_W = 800.0
_LOG_MAX = math.log(1000.0 / 16.0)

_NPAD = 7680
_ROWS = _NPAD // 128
_OROWS = 1024 // 128

_INT_MIN = np.int32(-2147483648)


def _sortable_key(score):
    b = lax.bitcast_convert_type(score, jnp.int32)
    return jnp.where(b >= 0, b, _INT_MIN - b)


def _top2000_mask(score, idx):
    key = _sortable_key(score)
    hi16 = jnp.right_shift(key, 16)
    lo16 = jnp.bitwise_and(key, jnp.int32(0xFFFF))

    h = jnp.int32(-32768)
    for bit in range(15, -1, -1):
        trial = h + jnp.int32(1 << bit)
        c = jnp.sum((hi16 >= trial).astype(jnp.int32))
        h = jnp.where(c >= _NUM_PRE, trial, h)

    g = jnp.sum((hi16 > h).astype(jnp.int32))
    m1 = jnp.int32(_NUM_PRE) - g
    eq_hi = hi16 == h
    l = jnp.int32(0)
    for bit in range(15, -1, -1):
        trial = l + jnp.int32(1 << bit)
        c = jnp.sum((eq_hi & (lo16 >= trial)).astype(jnp.int32))
        l = jnp.where(c >= m1, trial, l)

    kstar = jnp.bitwise_or(lax.shift_left(h, 16), l)
    gt = jnp.sum((key > kstar).astype(jnp.int32))
    m2 = jnp.int32(_NUM_PRE) - gt
    eq = key == kstar
    t = jnp.int32(0)
    for bit in range(12, -1, -1):
        trial = t + jnp.int32(1 << bit)
        c = jnp.sum((eq & (idx < trial)).astype(jnp.int32))
        t = jnp.where(c < m2, trial, t)

    return (key > kstar) | (eq & (idx <= t))


def _rpn_kernel(score_ref, regs_ref, anch_ref, out_ref, bsc_ref):
    idx = lax.broadcasted_iota(jnp.int32, (_ROWS, 128), 0) * 128 + \
        lax.broadcasted_iota(jnp.int32, (_ROWS, 128), 1)
    oidx = lax.broadcasted_iota(jnp.int32, (_OROWS, 128), 0) * 128 + \
        lax.broadcasted_iota(jnp.int32, (_OROWS, 128), 1)
    zero_plane = jnp.zeros((_OROWS, 128), jnp.float32)

    s0 = []
    boxes = []
    for b in range(2):
        score = score_ref[b]
        dx = regs_ref[b, 0]
        dy = regs_ref[b, 1]
        dh = jnp.minimum(regs_ref[b, 2], _LOG_MAX)
        dw = jnp.minimum(regs_ref[b, 3], _LOG_MAX)
        ax1 = anch_ref[b, 0]
        ay1 = anch_ref[b, 1]
        ax2 = anch_ref[b, 2]
        ay2 = anch_ref[b, 3]

        sel = _top2000_mask(score, idx)

        ah = ay2 - ay1
        aw = ax2 - ax1
        px = aw * 0.5 + dx * aw
        py = ah * 0.5 + dy * ah
        ph = jnp.exp(dh) * ah
        pw = jnp.exp(dw) * aw
        bx1 = px - pw * 0.5
        by1 = py - ph * 0.5
        bx2 = px + pw * 0.5
        by2 = py + ph * 0.5

        cminx = jnp.clip(bx1, 0.0, _IMG_W)
        cminy = jnp.clip(by1, 0.0, _IMG_H)
        cmaxx = jnp.clip(bx2, 0.0, _IMG_W)
        cmaxy = jnp.clip(by2, 0.0, _IMG_H)
        size_ok = ((cmaxx - cminx) >= _MIN_SIZE) & ((cmaxy - cminy) >= _MIN_SIZE)

        s0.append(jnp.where(sel & size_ok, score, -1e9))
        area = (bx2 - bx1) * (by2 - by1)
        boxes.append((bx1, by1, bx2, by2, area))
        bsc_ref[b, 0] = bx1
        bsc_ref[b, 1] = by1
        bsc_ref[b, 2] = bx2
        bsc_ref[b, 3] = by2

    lanevec = lax.broadcasted_iota(jnp.int32, (1, 128), 1)

    def peak(s):
        m = jnp.max(s)
        j = jnp.min(jnp.where(s == m, idx, jnp.int32(1 << 30)))
        return m, j

    def cond(state):
        i, ma, _, mb, _, _, _, _ = state
        return (i < _NUM_POST) & ((ma > -1e8) | (mb > -1e8))

    def step(i, m, j, s, outs, bx, b):
        bx1, by1, bx2, by2, area = bx
        valid = m > -1e8
        onehot = idx == j
        row = lax.shift_right_logical(j, 7)
        lane = jnp.bitwise_and(j, jnp.int32(127))
        lmask = lanevec == lane
        jx1 = jnp.sum(jnp.where(lmask, bsc_ref[b, 0, pl.ds(row, 1), :], 0.0))
        jy1 = jnp.sum(jnp.where(lmask, bsc_ref[b, 1, pl.ds(row, 1), :], 0.0))
        jx2 = jnp.sum(jnp.where(lmask, bsc_ref[b, 2, pl.ds(row, 1), :], 0.0))
        jy2 = jnp.sum(jnp.where(lmask, bsc_ref[b, 3, pl.ds(row, 1), :], 0.0))
        jarea = (jx2 - jx1) * (jy2 - jy1)
        iw = jnp.maximum(jnp.minimum(jx2, bx2) - jnp.maximum(jx1, bx1), 0.0)
        ih = jnp.maximum(jnp.minimum(jy2, by2) - jnp.maximum(jy1, by1), 0.0)
        inter = iw * ih
        iou = inter / (jarea + area - inter + 1e-9)
        s = jnp.where(((iou > _IOU_THR) | onehot) & valid, -1e9, s)
        osel = (oidx == i) & valid
        o1, o2, o3, o4 = outs
        o1 = jnp.where(osel, jx1, o1)
        o2 = jnp.where(osel, jy1, o2)
        o3 = jnp.where(osel, jx2, o3)
        o4 = jnp.where(osel, jy2, o4)
        m2, j2 = peak(s)
        return m2, j2, s, (o1, o2, o3, o4)

    def body(state):
        i, ma, ja, mb, jb, sa, sb, (oa, ob) = state
        ma, ja, sa, oa = step(i, ma, ja, sa, oa, boxes[0], 0)
        mb, jb, sb, ob = step(i, mb, jb, sb, ob, boxes[1], 1)
        return (i + 1, ma, ja, mb, jb, sa, sb, (oa, ob))

    ma0, ja0 = peak(s0[0])
    mb0, jb0 = peak(s0[1])
    zouts = (zero_plane, zero_plane, zero_plane, zero_plane)
    init = (jnp.int32(0), ma0, ja0, mb0, jb0, s0[0], s0[1], (zouts, zouts))
    final = lax.while_loop(cond, body, init)
    (oa1, oa2, oa3, oa4), (ob1, ob2, ob3, ob4) = final[7]
    out_ref[0, 0] = oa1
    out_ref[0, 1] = oa2
    out_ref[0, 2] = oa3
    out_ref[0, 3] = oa4
    out_ref[1, 0] = ob1
    out_ref[1, 1] = ob2
    out_ref[1, 2] = ob3
    out_ref[1, 3] = ob4


@jax.jit
def kernel(cls, regs, anchors):
    B = cls.shape[0]
    score = jnp.transpose(cls, (0, 2, 3, 1)).reshape(B, -1)
    regs_f = jnp.transpose(regs, (0, 2, 3, 1)).reshape(B, -1, 4)

    pad = _NPAD - score.shape[1]
    score_p = jnp.pad(score, ((0, 0), (0, pad)),
                      constant_values=-jnp.inf).reshape(B, _ROWS, 128)
    regs_p = jnp.moveaxis(jnp.pad(regs_f, ((0, 0), (0, pad), (0, 0))), 2, 1)
    regs_p = regs_p.reshape(B, 4, _ROWS, 128)
    anch_p = jnp.moveaxis(jnp.pad(anchors, ((0, 0), (0, pad), (0, 0))), 2, 1)
    anch_p = anch_p.reshape(B, 4, _ROWS, 128)

    out = pl.pallas_call(
        _rpn_kernel,
        out_shape=jax.ShapeDtypeStruct((B, 4, _OROWS, 128), jnp.float32),
        scratch_shapes=[pltpu.VMEM((B, 4, _ROWS, 128), jnp.float32)],
    )(score_p, regs_p, anch_p)

    boxes = jnp.moveaxis(out.reshape(B, 4, _OROWS * 128), 1, 2)
    return boxes[:, :_NUM_POST, :].reshape(B * _NUM_POST, 4)


# speedup vs baseline: 26.7335x; 1.3246x over previous
import math

import numpy as np
import jax
import jax.numpy as jnp
from jax import lax
from jax.experimental import pallas as pl
from jax.experimental.pallas import tpu as pltpu

_NUM_PRE = 2000
_NUM_POST = 1000
_IOU_THR = 0.7
_MIN_SIZE = 16.0
_IMG_H = 800.0
_IMG_W = 800.0
_LOG_MAX = math.log(1000.0 / 16.0)

_NPAD = 8192
_ROWS = _NPAD // 128
_CROWS = 2048 // 128
_OROWS = 1024 // 128

_INT_MIN = np.int32(-2147483648)


def _roll(x, shift, axis):
    return pltpu.roll(x, shift, axis)


def _sortable_key(score):
    b = lax.bitcast_convert_type(score, jnp.int32)
    return jnp.where(b >= 0, b, _INT_MIN - b)


def _rpn_kernel(score_ref, regs_ref, anch_ref, out_ref, bsc_ref):
    idx = lax.broadcasted_iota(jnp.int32, (_ROWS, 128), 0) * 128 + \
        lax.broadcasted_iota(jnp.int32, (_ROWS, 128), 1)
    slot = lax.broadcasted_iota(jnp.int32, (_CROWS, 128), 0) * 128 + \
        lax.broadcasted_iota(jnp.int32, (_CROWS, 128), 1)
    oidx = lax.broadcasted_iota(jnp.int32, (_OROWS, 128), 0) * 128 + \
        lax.broadcasted_iota(jnp.int32, (_OROWS, 128), 1)
    zero_plane = jnp.zeros((_OROWS, 128), jnp.float32)

    planes = []
    for b in range(2):
        score = score_ref[b]
        dx = regs_ref[b, 0]
        dy = regs_ref[b, 1]
        dh = jnp.minimum(regs_ref[b, 2], _LOG_MAX)
        dw = jnp.minimum(regs_ref[b, 3], _LOG_MAX)
        ax1 = anch_ref[b, 0]
        ay1 = anch_ref[b, 1]
        ax2 = anch_ref[b, 2]
        ay2 = anch_ref[b, 3]

        ah = ay2 - ay1
        aw = ax2 - ax1
        px = aw * 0.5 + dx * aw
        py = ah * 0.5 + dy * ah
        ph = jnp.exp(dh) * ah
        pw = jnp.exp(dw) * aw
        bx1 = px - pw * 0.5
        by1 = py - ph * 0.5
        bx2 = px + pw * 0.5
        by2 = py + ph * 0.5

        cminx = jnp.clip(bx1, 0.0, _IMG_W)
        cminy = jnp.clip(by1, 0.0, _IMG_H)
        cmaxx = jnp.clip(bx2, 0.0, _IMG_W)
        cmaxy = jnp.clip(by2, 0.0, _IMG_H)
        size_ok = ((cmaxx - cminx) >= _MIN_SIZE) & ((cmaxy - cminy) >= _MIN_SIZE)
        sm = jnp.where(size_ok, score, -1e9)

        planes.append([_sortable_key(score), idx, sm, bx1, by1, bx2, by2])

    def partner(x, j):
        if j < 128:
            return jnp.where((idx & j) == 0,
                             _roll(x, 128 - j, 1), _roll(x, j, 1))
        r = j // 128
        return jnp.where((idx & j) == 0,
                         _roll(x, _ROWS - r, 0), _roll(x, r, 0))

    k = 2
    while k <= _NPAD:
        j = k // 2
        while j >= 1:
            up = (idx & k) == 0
            is_lo = (idx & j) == 0
            take_small = up == is_lo
            for b in range(2):
                K, I = planes[b][0], planes[b][1]
                Kp, Ip = partner(K, j), partner(I, j)
                sl = (K > Kp) | ((K == Kp) & (I < Ip))
                ch = take_small == sl
                planes[b] = [jnp.where(ch, p, partner(p, j))
                             for p in planes[b]]
            j //= 2
        k *= 2

    s0 = []
    boxes = []
    for b in range(2):
        _, _, sm, bx1, by1, bx2, by2 = [
            lax.slice(p, (0, 0), (_CROWS, 128)) for p in planes[b]]
        s0.append(jnp.where(slot < _NUM_PRE, sm, -1e9))
        area = (bx2 - bx1) * (by2 - by1)
        boxes.append((bx1, by1, bx2, by2, area))
        bsc_ref[b, 0] = bx1
        bsc_ref[b, 1] = by1
        bsc_ref[b, 2] = bx2
        bsc_ref[b, 3] = by2

    lanevec = lax.broadcasted_iota(jnp.int32, (1, 128), 1)
    big = jnp.int32(1 << 30)

    def peak(s):
        return jnp.min(jnp.where(s > -1e8, slot, big))

    def cond(state):
        i, ja, jb, _, _, _ = state
        return (i < _NUM_POST) & ((ja < big) | (jb < big))

    def step(i, j, s, outs, bx, b):
        bx1, by1, bx2, by2, area = bx
        valid = j < big
        jc = jnp.where(valid, j, 0)
        row = lax.shift_right_logical(jc, 7)
        lane = jnp.bitwise_and(jc, jnp.int32(127))
        lmask = lanevec == lane
        jx1 = jnp.sum(jnp.where(lmask, bsc_ref[b, 0, pl.ds(row, 1), :], 0.0))
        jy1 = jnp.sum(jnp.where(lmask, bsc_ref[b, 1, pl.ds(row, 1), :], 0.0))
        jx2 = jnp.sum(jnp.where(lmask, bsc_ref[b, 2, pl.ds(row, 1), :], 0.0))
        jy2 = jnp.sum(jnp.where(lmask, bsc_ref[b, 3, pl.ds(row, 1), :], 0.0))
        jarea = (jx2 - jx1) * (jy2 - jy1)
        iw = jnp.maximum(jnp.minimum(jx2, bx2) - jnp.maximum(jx1, bx1), 0.0)
        ih = jnp.maximum(jnp.minimum(jy2, by2) - jnp.maximum(jy1, by1), 0.0)
        inter = iw * ih
        iou = inter / (jarea + area - inter + 1e-9)
        s = jnp.where(((iou > _IOU_THR) | (slot == j)) & valid, -1e9, s)
        osel = (oidx == i) & valid
        o1, o2, o3, o4 = outs
        o1 = jnp.where(osel, jx1, o1)
        o2 = jnp.where(osel, jy1, o2)
        o3 = jnp.where(osel, jx2, o3)
        o4 = jnp.where(osel, jy2, o4)
        return peak(s), s, (o1, o2, o3, o4)

    def body(state):
        i, ja, jb, sa, sb, (oa, ob) = state
        ja, sa, oa = step(i, ja, sa, oa, boxes[0], 0)
        jb, sb, ob = step(i, jb, sb, ob, boxes[1], 1)
        return (i + 1, ja, jb, sa, sb, (oa, ob))

    zouts = (zero_plane, zero_plane, zero_plane, zero_plane)
    init = (jnp.int32(0), peak(s0[0]), peak(s0[1]), s0[0], s0[1],
            (zouts, zouts))
    final = lax.while_loop(cond, body, init)
    (oa1, oa2, oa3, oa4), (ob1, ob2, ob3, ob4) = final[5]
    out_ref[0, 0] = oa1
    out_ref[0, 1] = oa2
    out_ref[0, 2] = oa3
    out_ref[0, 3] = oa4
    out_ref[1, 0] = ob1
    out_ref[1, 1] = ob2
    out_ref[1, 2] = ob3
    out_ref[1, 3] = ob4


@jax.jit
def kernel(cls, regs, anchors):
    B = cls.shape[0]
    score = jnp.transpose(cls, (0, 2, 3, 1)).reshape(B, -1)
    regs_f = jnp.transpose(regs, (0, 2, 3, 1)).reshape(B, -1, 4)

    pad = _NPAD - score.shape[1]
    score_p = jnp.pad(score, ((0, 0), (0, pad)),
                      constant_values=-jnp.inf).reshape(B, _ROWS, 128)
    regs_p = jnp.moveaxis(jnp.pad(regs_f, ((0, 0), (0, pad), (0, 0))), 2, 1)
    regs_p = regs_p.reshape(B, 4, _ROWS, 128)
    anch_p = jnp.moveaxis(jnp.pad(anchors, ((0, 0), (0, pad), (0, 0))), 2, 1)
    anch_p = anch_p.reshape(B, 4, _ROWS, 128)

    out = pl.pallas_call(
        _rpn_kernel,
        out_shape=jax.ShapeDtypeStruct((B, 4, _OROWS, 128), jnp.float32),
        scratch_shapes=[pltpu.VMEM((B, 4, _CROWS, 128), jnp.float32)],
    )(score_p, regs_p, anch_p)

    boxes = jnp.moveaxis(out.reshape(B, 4, _OROWS * 128), 1, 2)
    return boxes[:, :_NUM_POST, :].reshape(B * _NUM_POST, 4)


# speedup vs baseline: 29.3172x; 1.0966x over previous
import math

import numpy as np
import jax
import jax.numpy as jnp
from jax import lax
from jax.experimental import pallas as pl
from jax.experimental.pallas import tpu as pltpu

_NUM_PRE = 2000
_NUM_POST = 1000
_IOU_THR = 0.7
_MIN_SIZE = 16.0
_IMG_H = 800.0
_IMG_W = 800.0
_LOG_MAX = math.log(1000.0 / 16.0)

_NPAD = 8192
_ROWS = _NPAD // 128
_CROWS = 2048 // 128
_OROWS = 1024 // 128

_INT_MIN = np.int32(-2147483648)


def _roll(x, shift, axis):
    return pltpu.roll(x, shift, axis)


def _sortable_key(score):
    b = lax.bitcast_convert_type(score, jnp.int32)
    return jnp.where(b >= 0, b, _INT_MIN - b)


def _rpn_kernel(score_ref, regs_ref, anch_ref, out_ref, bsc_ref):
    idx = lax.broadcasted_iota(jnp.int32, (_ROWS, 128), 0) * 128 + \
        lax.broadcasted_iota(jnp.int32, (_ROWS, 128), 1)
    slot = lax.broadcasted_iota(jnp.int32, (_CROWS, 128), 0) * 128 + \
        lax.broadcasted_iota(jnp.int32, (_CROWS, 128), 1)
    oidx = lax.broadcasted_iota(jnp.int32, (_OROWS, 128), 0) * 128 + \
        lax.broadcasted_iota(jnp.int32, (_OROWS, 128), 1)
    zero_plane = jnp.zeros((_OROWS, 128), jnp.float32)

    planes = []
    for b in range(2):
        score = score_ref[b]
        dx = regs_ref[b, 0]
        dy = regs_ref[b, 1]
        dh = jnp.minimum(regs_ref[b, 2], _LOG_MAX)
        dw = jnp.minimum(regs_ref[b, 3], _LOG_MAX)
        ax1 = anch_ref[b, 0]
        ay1 = anch_ref[b, 1]
        ax2 = anch_ref[b, 2]
        ay2 = anch_ref[b, 3]

        ah = ay2 - ay1
        aw = ax2 - ax1
        px = aw * 0.5 + dx * aw
        py = ah * 0.5 + dy * ah
        ph = jnp.exp(dh) * ah
        pw = jnp.exp(dw) * aw
        bx1 = px - pw * 0.5
        by1 = py - ph * 0.5
        bx2 = px + pw * 0.5
        by2 = py + ph * 0.5

        cminx = jnp.clip(bx1, 0.0, _IMG_W)
        cminy = jnp.clip(by1, 0.0, _IMG_H)
        cmaxx = jnp.clip(bx2, 0.0, _IMG_W)
        cmaxy = jnp.clip(by2, 0.0, _IMG_H)
        size_ok = ((cmaxx - cminx) >= _MIN_SIZE) & ((cmaxy - cminy) >= _MIN_SIZE)
        sm = jnp.where(size_ok, score, -1e9)

        planes.append([_sortable_key(score), idx, sm, bx1, by1, bx2, by2])

    def partner(x, j):
        if j < 128:
            return jnp.where((idx & j) == 0,
                             _roll(x, 128 - j, 1), _roll(x, j, 1))
        r = j // 128
        return jnp.where((idx & j) == 0,
                         _roll(x, _ROWS - r, 0), _roll(x, r, 0))

    k = 2
    while k <= _NPAD:
        j = k // 2
        while j >= 1:
            up = (idx & k) == 0
            is_lo = (idx & j) == 0
            take_small = up == is_lo
            for b in range(2):
                K, I = planes[b][0], planes[b][1]
                Kp, Ip = partner(K, j), partner(I, j)
                sl = (K > Kp) | ((K == Kp) & (I < Ip))
                ch = take_small == sl
                planes[b] = [jnp.where(ch, p, partner(p, j))
                             for p in planes[b]]
            j //= 2
        k *= 2

    s0 = []
    boxes = []
    for b in range(2):
        _, _, sm, bx1, by1, bx2, by2 = [
            lax.slice(p, (0, 0), (_CROWS, 128)) for p in planes[b]]
        s0.append(jnp.where(slot < _NUM_PRE, sm, -1e9))
        area = (bx2 - bx1) * (by2 - by1)
        boxes.append((bx1, by1, bx2, by2, area))
        bsc_ref[b, 0] = bx1
        bsc_ref[b, 1] = by1
        bsc_ref[b, 2] = bx2
        bsc_ref[b, 3] = by2

    lanevec = lax.broadcasted_iota(jnp.int32, (1, 128), 1)
    big = jnp.int32(1 << 30)

    def peak(s):
        return jnp.min(jnp.where(s > -1e8, slot, big))

    def cond(state):
        i, ja, jb, _, _, _ = state
        return (i < _NUM_POST // 2) & ((ja < big) | (jb < big))

    def step(i, j, s, outs, bx, b):
        bx1, by1, bx2, by2, area = bx
        valid = j < big
        jc = jnp.where(valid, j, 0)
        row = lax.shift_right_logical(jc, 7)
        lane = jnp.bitwise_and(jc, jnp.int32(127))
        lmask = lanevec == lane
        jx1 = jnp.sum(jnp.where(lmask, bsc_ref[b, 0, pl.ds(row, 1), :], 0.0))
        jy1 = jnp.sum(jnp.where(lmask, bsc_ref[b, 1, pl.ds(row, 1), :], 0.0))
        jx2 = jnp.sum(jnp.where(lmask, bsc_ref[b, 2, pl.ds(row, 1), :], 0.0))
        jy2 = jnp.sum(jnp.where(lmask, bsc_ref[b, 3, pl.ds(row, 1), :], 0.0))
        jarea = (jx2 - jx1) * (jy2 - jy1)
        iw = jnp.maximum(jnp.minimum(jx2, bx2) - jnp.maximum(jx1, bx1), 0.0)
        ih = jnp.maximum(jnp.minimum(jy2, by2) - jnp.maximum(jy1, by1), 0.0)
        inter = iw * ih
        iou = inter / (jarea + area - inter + 1e-9)
        s = jnp.where(((iou > _IOU_THR) | (slot == j)) & valid, -1e9, s)
        osel = (oidx == i) & valid
        o1, o2, o3, o4 = outs
        o1 = jnp.where(osel, jx1, o1)
        o2 = jnp.where(osel, jy1, o2)
        o3 = jnp.where(osel, jx2, o3)
        o4 = jnp.where(osel, jy2, o4)
        return peak(s), s, (o1, o2, o3, o4)

    def body(state):
        i, ja, jb, sa, sb, (oa, ob) = state
        i2 = i * 2
        ja, sa, oa = step(i2, ja, sa, oa, boxes[0], 0)
        jb, sb, ob = step(i2, jb, sb, ob, boxes[1], 1)
        ja, sa, oa = step(i2 + 1, ja, sa, oa, boxes[0], 0)
        jb, sb, ob = step(i2 + 1, jb, sb, ob, boxes[1], 1)
        return (i + 1, ja, jb, sa, sb, (oa, ob))

    zouts = (zero_plane, zero_plane, zero_plane, zero_plane)
    init = (jnp.int32(0), peak(s0[0]), peak(s0[1]), s0[0], s0[1],
            (zouts, zouts))
    final = lax.while_loop(cond, body, init)
    (oa1, oa2, oa3, oa4), (ob1, ob2, ob3, ob4) = final[5]
    out_ref[0, 0] = oa1
    out_ref[0, 1] = oa2
    out_ref[0, 2] = oa3
    out_ref[0, 3] = oa4
    out_ref[1, 0] = ob1
    out_ref[1, 1] = ob2
    out_ref[1, 2] = ob3
    out_ref[1, 3] = ob4


@jax.jit
def kernel(cls, regs, anchors):
    B = cls.shape[0]
    score = jnp.transpose(cls, (0, 2, 3, 1)).reshape(B, -1)
    regs_f = jnp.transpose(regs, (0, 2, 3, 1)).reshape(B, -1, 4)

    pad = _NPAD - score.shape[1]
    score_p = jnp.pad(score, ((0, 0), (0, pad)),
                      constant_values=-jnp.inf).reshape(B, _ROWS, 128)
    regs_p = jnp.moveaxis(jnp.pad(regs_f, ((0, 0), (0, pad), (0, 0))), 2, 1)
    regs_p = regs_p.reshape(B, 4, _ROWS, 128)
    anch_p = jnp.moveaxis(jnp.pad(anchors, ((0, 0), (0, pad), (0, 0))), 2, 1)
    anch_p = anch_p.reshape(B, 4, _ROWS, 128)

    out = pl.pallas_call(
        _rpn_kernel,
        out_shape=jax.ShapeDtypeStruct((B, 4, _OROWS, 128), jnp.float32),
        scratch_shapes=[pltpu.VMEM((B, 4, _CROWS, 128), jnp.float32)],
    )(score_p, regs_p, anch_p)

    boxes = jnp.moveaxis(out.reshape(B, 4, _OROWS * 128), 1, 2)
    return boxes[:, :_NUM_POST, :].reshape(B * _NUM_POST, 4)


# speedup vs baseline: 30.7141x; 1.0476x over previous
import math

import numpy as np
import jax
import jax.numpy as jnp
from jax import lax
from jax.experimental import pallas as pl
from jax.experimental.pallas import tpu as pltpu

_NUM_PRE = 2000
_NUM_POST = 1000
_IOU_THR = 0.7
_MIN_SIZE = 16.0
_IMG_H = 800.0
_IMG_W = 800.0
_LOG_MAX = math.log(1000.0 / 16.0)

_NPAD = 8192
_ROWS = _NPAD // 128
_CROWS = 2048 // 128
_OROWS = 1024 // 128

_INT_MIN = np.int32(-2147483648)


def _roll(x, shift, axis):
    return pltpu.roll(x, shift, axis)


def _sortable_key(score):
    b = lax.bitcast_convert_type(score, jnp.int32)
    return jnp.where(b >= 0, b, _INT_MIN - b)


def _rpn_kernel(score_ref, regs_ref, anch_ref, out_ref, bsc_ref):
    idx = lax.broadcasted_iota(jnp.int32, (_ROWS, 128), 0) * 128 + \
        lax.broadcasted_iota(jnp.int32, (_ROWS, 128), 1)
    slot = lax.broadcasted_iota(jnp.int32, (_CROWS, 128), 0) * 128 + \
        lax.broadcasted_iota(jnp.int32, (_CROWS, 128), 1)
    oidx = lax.broadcasted_iota(jnp.int32, (_OROWS, 128), 0) * 128 + \
        lax.broadcasted_iota(jnp.int32, (_OROWS, 128), 1)
    zero_plane = jnp.zeros((_OROWS, 128), jnp.float32)

    planes = []
    for b in range(2):
        score = score_ref[b]
        dx = regs_ref[b, 0]
        dy = regs_ref[b, 1]
        dh = jnp.minimum(regs_ref[b, 2], _LOG_MAX)
        dw = jnp.minimum(regs_ref[b, 3], _LOG_MAX)
        ax1 = anch_ref[b, 0]
        ay1 = anch_ref[b, 1]
        ax2 = anch_ref[b, 2]
        ay2 = anch_ref[b, 3]

        ah = ay2 - ay1
        aw = ax2 - ax1
        px = aw * 0.5 + dx * aw
        py = ah * 0.5 + dy * ah
        ph = jnp.exp(dh) * ah
        pw = jnp.exp(dw) * aw
        bx1 = px - pw * 0.5
        by1 = py - ph * 0.5
        bx2 = px + pw * 0.5
        by2 = py + ph * 0.5

        cminx = jnp.clip(bx1, 0.0, _IMG_W)
        cminy = jnp.clip(by1, 0.0, _IMG_H)
        cmaxx = jnp.clip(bx2, 0.0, _IMG_W)
        cmaxy = jnp.clip(by2, 0.0, _IMG_H)
        size_ok = ((cmaxx - cminx) >= _MIN_SIZE) & ((cmaxy - cminy) >= _MIN_SIZE)
        sm = jnp.where(size_ok, score, -1e9)

        planes.append([_sortable_key(score), idx, sm, bx1, by1, bx2, by2])

    def partner(x, j):
        if j < 128:
            return jnp.where((idx & j) == 0,
                             _roll(x, 128 - j, 1), _roll(x, j, 1))
        r = j // 128
        return jnp.where((idx & j) == 0,
                         _roll(x, _ROWS - r, 0), _roll(x, r, 0))

    k = 2
    while k <= _NPAD:
        j = k // 2
        while j >= 1:
            up = (idx & k) == 0
            is_lo = (idx & j) == 0
            take_small = up == is_lo
            for b in range(2):
                K, I = planes[b][0], planes[b][1]
                Kp, Ip = partner(K, j), partner(I, j)
                sl = (K > Kp) | ((K == Kp) & (I < Ip))
                ch = take_small == sl
                planes[b] = [jnp.where(ch, p, partner(p, j))
                             for p in planes[b]]
            j //= 2
        k *= 2

    s0 = []
    boxes = []
    for b in range(2):
        _, _, sm, bx1, by1, bx2, by2 = [
            lax.slice(p, (0, 0), (_CROWS, 128)) for p in planes[b]]
        s0.append(jnp.where(slot < _NUM_PRE, sm, -1e9))
        area = (bx2 - bx1) * (by2 - by1)
        boxes.append((bx1, by1, bx2, by2, area))
        bsc_ref[b, 0] = bx1
        bsc_ref[b, 1] = by1
        bsc_ref[b, 2] = bx2
        bsc_ref[b, 3] = by2

    lanevec = lax.broadcasted_iota(jnp.int32, (1, 128), 1)
    big = jnp.int32(1 << 30)

    def peak(s):
        return jnp.min(jnp.where(s > -1e8, slot, big))

    def cond(state):
        i, ja, jb, _, _, _ = state
        return (i < _NUM_POST // 4) & ((ja < big) | (jb < big))

    def step(i, j, s, outs, bx, b):
        bx1, by1, bx2, by2, area = bx
        valid = j < big
        jc = jnp.where(valid, j, 0)
        row = lax.shift_right_logical(jc, 7)
        lane = jnp.bitwise_and(jc, jnp.int32(127))
        lmask = lanevec == lane
        jx1 = jnp.sum(jnp.where(lmask, bsc_ref[b, 0, pl.ds(row, 1), :], 0.0))
        jy1 = jnp.sum(jnp.where(lmask, bsc_ref[b, 1, pl.ds(row, 1), :], 0.0))
        jx2 = jnp.sum(jnp.where(lmask, bsc_ref[b, 2, pl.ds(row, 1), :], 0.0))
        jy2 = jnp.sum(jnp.where(lmask, bsc_ref[b, 3, pl.ds(row, 1), :], 0.0))
        jarea = (jx2 - jx1) * (jy2 - jy1)
        iw = jnp.maximum(jnp.minimum(jx2, bx2) - jnp.maximum(jx1, bx1), 0.0)
        ih = jnp.maximum(jnp.minimum(jy2, by2) - jnp.maximum(jy1, by1), 0.0)
        inter = iw * ih
        iou = inter / (jarea + area - inter + 1e-9)
        s = jnp.where(((iou > _IOU_THR) | (slot == j)) & valid, -1e9, s)
        osel = (oidx == i) & valid
        o1, o2, o3, o4 = outs
        o1 = jnp.where(osel, jx1, o1)
        o2 = jnp.where(osel, jy1, o2)
        o3 = jnp.where(osel, jx2, o3)
        o4 = jnp.where(osel, jy2, o4)
        return peak(s), s, (o1, o2, o3, o4)

    def body(state):
        i, ja, jb, sa, sb, (oa, ob) = state
        i4 = i * 4
        for u in range(4):
            ja, sa, oa = step(i4 + u, ja, sa, oa, boxes[0], 0)
            jb, sb, ob = step(i4 + u, jb, sb, ob, boxes[1], 1)
        return (i + 1, ja, jb, sa, sb, (oa, ob))

    zouts = (zero_plane, zero_plane, zero_plane, zero_plane)
    init = (jnp.int32(0), peak(s0[0]), peak(s0[1]), s0[0], s0[1],
            (zouts, zouts))
    final = lax.while_loop(cond, body, init)
    (oa1, oa2, oa3, oa4), (ob1, ob2, ob3, ob4) = final[5]
    out_ref[0, 0] = oa1
    out_ref[0, 1] = oa2
    out_ref[0, 2] = oa3
    out_ref[0, 3] = oa4
    out_ref[1, 0] = ob1
    out_ref[1, 1] = ob2
    out_ref[1, 2] = ob3
    out_ref[1, 3] = ob4


@jax.jit
def kernel(cls, regs, anchors):
    B = cls.shape[0]
    score = jnp.transpose(cls, (0, 2, 3, 1)).reshape(B, -1)
    regs_f = jnp.transpose(regs, (0, 2, 3, 1)).reshape(B, -1, 4)

    pad = _NPAD - score.shape[1]
    score_p = jnp.pad(score, ((0, 0), (0, pad)),
                      constant_values=-jnp.inf).reshape(B, _ROWS, 128)
    regs_p = jnp.moveaxis(jnp.pad(regs_f, ((0, 0), (0, pad), (0, 0))), 2, 1)
    regs_p = regs_p.reshape(B, 4, _ROWS, 128)
    anch_p = jnp.moveaxis(jnp.pad(anchors, ((0, 0), (0, pad), (0, 0))), 2, 1)
    anch_p = anch_p.reshape(B, 4, _ROWS, 128)

    out = pl.pallas_call(
        _rpn_kernel,
        out_shape=jax.ShapeDtypeStruct((B, 4, _OROWS, 128), jnp.float32),
        scratch_shapes=[pltpu.VMEM((B, 4, _CROWS, 128), jnp.float32)],
    )(score_p, regs_p, anch_p)

    boxes = jnp.moveaxis(out.reshape(B, 4, _OROWS * 128), 1, 2)
    return boxes[:, :_NUM_POST, :].reshape(B * _NUM_POST, 4)
